# Initial kernel scaffold; baseline (speedup 1.0000x reference)
import jax
import jax.numpy as jnp
from jax.experimental import pallas as pl


def kernel(page_table_dst, page_table_a, page_table_b, seq_len_a, seq_len_b):
    raise NotImplementedError("write your pallas kernel here")


# speedup vs baseline: 178.4698x; 178.4698x over previous
import jax
import jax.numpy as jnp
from jax.experimental import pallas as pl
from jax.experimental.pallas import tpu as pltpu

K = 4
ROWS_PER_BLK = 8
LEN_A = 4096
LEN_B = 64
LEN_DST = LEN_A + LEN_B


def _blend_kernel(la_s, lb_s, dst_ref, a_ref, b_ref, la_v, lb_v, out_ref):
    i = pl.program_id(0)
    cols = jax.lax.broadcasted_iota(jnp.int32, (ROWS_PER_BLK, LEN_DST), 1)
    la = la_v[...]
    a2 = jnp.squeeze(a_ref[...], axis=1)
    a_exp = jnp.concatenate(
        [a2[0:1]] * K + [a2[1:2]] * K, axis=0)
    a_pad = jnp.concatenate(
        [a_exp, jnp.zeros((ROWS_PER_BLK, LEN_B), a_exp.dtype)], axis=1)
    out_ref[...] = jnp.where(cols < la, a_pad, dst_ref[...])
    for r in range(ROWS_PER_BLK):
        row = i * ROWS_PER_BLK + r
        la_r = la_s[row]
        lb_r = lb_s[row]
        bp = jnp.concatenate(
            [b_ref[pl.ds(r, 1), :], jnp.zeros((1, 128 - LEN_B),
                                              b_ref.dtype)], axis=1)

        def blend(off, width):
            cols = jax.lax.broadcasted_iota(
                jnp.int32, (1, width), 1) + off
            seg = out_ref[pl.ds(r, 1), pl.ds(off, width)]
            bv = pltpu.roll(bp, (la_r - off) % 128, axis=1)[:, :width]
            m_b = (cols >= la_r) & (cols < la_r + lb_r)
            out_ref[pl.ds(r, 1), pl.ds(off, width)] = jnp.where(m_b, bv, seg)

        off0 = pl.multiple_of((la_r // 128) * 128, 128)
        blend(off0, 128)
        blend(pl.multiple_of(jnp.minimum(off0 + 128, LEN_A - 128), 128), 128)
        blend(LEN_A, LEN_B)


def kernel(page_table_dst, page_table_a, page_table_b, seq_len_a, seq_len_b):
    bs_expand = page_table_dst.shape[0]
    la_exp = jnp.repeat(seq_len_a.astype(jnp.int32), K)
    lb = seq_len_b.astype(jnp.int32)
    n_blk = bs_expand // ROWS_PER_BLK
    grid_spec = pltpu.PrefetchScalarGridSpec(
        num_scalar_prefetch=2,
        grid=(n_blk,),
        in_specs=[
            pl.BlockSpec((ROWS_PER_BLK, LEN_DST), lambda i, *_: (i, 0)),
            pl.BlockSpec((ROWS_PER_BLK // K, 1, LEN_A),
                         lambda i, *_: (i, 0, 0)),
            pl.BlockSpec((ROWS_PER_BLK, LEN_B), lambda i, *_: (i, 0)),
            pl.BlockSpec((ROWS_PER_BLK, 1), lambda i, *_: (i, 0)),
            pl.BlockSpec((ROWS_PER_BLK, 1), lambda i, *_: (i, 0)),
        ],
        out_specs=pl.BlockSpec((ROWS_PER_BLK, LEN_DST), lambda i, *_: (i, 0)),
    )
    return pl.pallas_call(
        _blend_kernel,
        grid_spec=grid_spec,
        out_shape=jax.ShapeDtypeStruct(page_table_dst.shape,
                                       page_table_dst.dtype),
    )(la_exp, lb, page_table_dst, page_table_a[:, None, :], page_table_b,
      la_exp[:, None], lb[:, None])
